# baseline (speedup 1.0000x reference)
import functools

import jax
import jax.numpy as jnp
from jax import lax
from jax.experimental import pallas as pl
from jax.experimental.pallas import tpu as pltpu
from jax.experimental.pallas import tpu_sc as plsc

N = 10000
E = 320000
D = 128
H = 128
OUT = 64

NC = 2
NS = 16
NW = NC * NS
CH = 128
KCH = 80
PH = 2
KPH = KCH // PH
SEG = 20
NCHUNK = NW * KCH
EPAD = NCHUNK * CH
NP = 10240
RPT = NP // NS

BLK = 640
GRID = NP // BLK


def _proj_body(x_ref, w_ref, b_ref, o_ref):
    h = jnp.dot(x_ref[...], w_ref[...], preferred_element_type=jnp.float32)
    o_ref[...] = jnp.maximum(h + b_ref[...], 0.0)


def _proj(x, w, b):
    return pl.pallas_call(
        _proj_body,
        grid=(GRID,),
        in_specs=[
            pl.BlockSpec((BLK, D), lambda i: (i, 0)),
            pl.BlockSpec((D, H), lambda i: (0, 0)),
            pl.BlockSpec((1, H), lambda i: (0, 0)),
        ],
        out_specs=pl.BlockSpec((BLK, H), lambda i: (i, 0)),
        out_shape=jax.ShapeDtypeStruct((NP, H), jnp.float32),
    )(x, w, b.reshape(1, H))


def _combine_body(k, *refs):
    s_refs = refs[0:k]
    c_refs = refs[k:2 * k]
    x_ref = refs[2 * k]
    w_refs = refs[2 * k + 1:3 * k + 1]
    r_refs = refs[3 * k + 1:4 * k + 1]
    b_refs = refs[4 * k + 1:5 * k + 1]
    nw_ref = refs[5 * k + 1]
    nb_ref = refs[5 * k + 2]
    o_ref = refs[5 * k + 3]

    h = jnp.zeros((BLK, H), jnp.float32)
    r = jnp.zeros((H, H), jnp.float32)
    for e in range(k):
        s = s_refs[e][0] + s_refs[e][1]
        c = jnp.sum(c_refs[e][...], axis=0)[:, None]
        a = s / jnp.maximum(c, 1.0)
        h = h + jnp.dot(a, w_refs[e][...], preferred_element_type=jnp.float32)
        h = h + b_refs[e][...]
        r = r + r_refs[e][...]
    h = (h + jnp.dot(x_ref[...], r, preferred_element_type=jnp.float32)) / float(k)
    mu = jnp.mean(h, axis=-1, keepdims=True)
    var = jnp.mean((h - mu) ** 2, axis=-1, keepdims=True)
    h = (h - mu) * lax.rsqrt(var + 1e-5) * nw_ref[...] + nb_ref[...]
    o_ref[...] = jnp.maximum(h, 0.0)


def _combine(sums, cnts, x, ws, rs, bs, nw, nb):
    k = len(sums)
    in_specs = (
        [pl.BlockSpec((NC, BLK, H), lambda i: (0, i, 0))] * k
        + [pl.BlockSpec((NW, BLK), lambda i: (0, i))] * k
        + [pl.BlockSpec((BLK, H), lambda i: (i, 0))]
        + [pl.BlockSpec((H, H), lambda i: (0, 0))] * (2 * k)
        + [pl.BlockSpec((1, H), lambda i: (0, 0))] * (k + 2)
    )
    args = (list(sums) + list(cnts) + [x] + list(ws) + list(rs)
            + [b.reshape(1, H) for b in bs] + [nw.reshape(1, H), nb.reshape(1, H)])
    return pl.pallas_call(
        functools.partial(_combine_body, k),
        grid=(GRID,),
        in_specs=in_specs,
        out_specs=pl.BlockSpec((BLK, H), lambda i: (i, 0)),
        out_shape=jax.ShapeDtypeStruct((NP, H), jnp.float32),
    )(*args)


def _final_body(x_ref, w_ref, b_ref, o_ref):
    h = jnp.dot(x_ref[...], w_ref[...], preferred_element_type=jnp.float32)
    o_ref[...] = h + b_ref[...]


def _final(x, w, b):
    return pl.pallas_call(
        _final_body,
        grid=(GRID,),
        in_specs=[
            pl.BlockSpec((BLK, H), lambda i: (i, 0)),
            pl.BlockSpec((H, OUT), lambda i: (0, 0)),
            pl.BlockSpec((1, OUT), lambda i: (0, 0)),
        ],
        out_specs=pl.BlockSpec((BLK, OUT), lambda i: (i, 0)),
        out_shape=jax.ShapeDtypeStruct((NP, OUT), jnp.float32),
    )(x, w, b.reshape(1, OUT))


@functools.cache
def _mesh():
    return plsc.VectorSubcoreMesh(
        core_axis_name="c", subcore_axis_name="s", num_cores=NC, num_subcores=NS)


def _agg3_body(xs_hbm, xd_hbm, s0_hbm, d0_hbm, s1_hbm, d1_hbm, s2_hbm, d2_hbm,
               zeros_hbm, o0_hbm, o1_hbm, o2_hbm,
               idx_s, idx_d, buf_a, buf_b, acc, sem_a, sem_b):
    cid = lax.axis_index("c")
    sid = lax.axis_index("s")
    wid = sid * NC + cid
    r0 = sid * RPT
    c0 = wid * KCH

    bufs = (buf_a, buf_b)
    sems = (sem_a, sem_b)
    rels = ((xs_hbm, s0_hbm, d0_hbm, o0_hbm),
            (xd_hbm, s1_hbm, d1_hbm, o1_hbm),
            (xs_hbm, s2_hbm, d2_hbm, o2_hbm))
    for x_hbm, src_hbm, dst_hbm, out_hbm in rels:
        pltpu.sync_copy(zeros_hbm, acc.at[pl.ds(r0, RPT)])
        plsc.subcore_barrier()

        for p in range(PH):
            base = c0 + p * KPH
            pltpu.sync_copy(src_hbm.at[pl.ds(base, KPH)], idx_s)
            pltpu.sync_copy(dst_hbm.at[pl.ds(base, KPH)], idx_d)

            def trip(seg, carry):
                b = seg * SEG
                cur = pltpu.async_copy(x_hbm.at[idx_s.at[b]], bufs[0], sems[0])
                for j in range(SEG):
                    nxt = None
                    if j < SEG - 1:
                        nxt = pltpu.async_copy(
                            x_hbm.at[idx_s.at[b + j + 1]],
                            bufs[(j + 1) % 2], sems[(j + 1) % 2])
                    cur.wait()
                    pltpu.sync_copy(bufs[j % 2], acc.at[idx_d.at[b + j]],
                                    add=True)
                    cur = nxt
                return carry

            lax.fori_loop(0, KPH // SEG, trip, 0)

        plsc.subcore_barrier()
        pltpu.sync_copy(acc.at[pl.ds(r0, RPT)], out_hbm.at[cid, pl.ds(r0, RPT)])


def _agg3(xs, xd, src0, dst0, src1, dst1, src2, dst2, zeros):
    return pl.kernel(
        _agg3_body,
        out_type=(jax.ShapeDtypeStruct((NC, NP, H), jnp.float32),) * 3,
        mesh=_mesh(),
        scratch_types=[
            pltpu.VMEM((KPH, CH), jnp.int32),
            pltpu.VMEM((KPH, CH), jnp.int32),
            pltpu.VMEM((CH, H), jnp.float32),
            pltpu.VMEM((CH, H), jnp.float32),
            pltpu.VMEM_SHARED((NP, H), jnp.float32),
            pltpu.SemaphoreType.DMA,
            pltpu.SemaphoreType.DMA,
        ],
    )(xs, xd, src0, dst0, src1, dst1, src2, dst2, zeros)


def _counts_body(d0_hbm, d1_hbm, d2_hbm, zeros_hbm, cnt_hbm, idx_d, cnt, sem):
    cid = lax.axis_index("c")
    sid = lax.axis_index("s")
    wid = sid * NC + cid
    c0 = wid * KCH
    ones = jnp.ones((16,), jnp.float32)

    for rel, d_hbm in enumerate((d0_hbm, d1_hbm, d2_hbm)):
        pltpu.sync_copy(zeros_hbm, cnt)
        pltpu.sync_copy(d_hbm.at[pl.ds(c0, KCH)], idx_d)

        def body(i, carry):
            for j in range(CH // 16):
                idx = idx_d[i, pl.ds(j * 16, 16)]
                plsc.addupdate_scatter(cnt, [idx], ones)
            return carry

        lax.fori_loop(0, KCH, body, 0)
        pltpu.sync_copy(cnt, cnt_hbm.at[pl.ds((rel * NW + wid) * NP, NP)])


def _counts(dst0, dst1, dst2, zeros1d):
    return pl.kernel(
        _counts_body,
        out_type=jax.ShapeDtypeStruct((3 * NW * NP,), jnp.float32),
        mesh=_mesh(),
        compiler_params=pltpu.CompilerParams(needs_layout_passes=False),
        scratch_types=[
            pltpu.VMEM((KCH, CH), jnp.int32),
            pltpu.VMEM((NP,), jnp.float32),
            pltpu.SemaphoreType.DMA,
        ],
    )(dst0, dst1, dst2, zeros1d)


def _pad_edges(ei):
    src = jnp.concatenate(
        [ei[0], jnp.zeros((EPAD - E,), jnp.int32)]).reshape(NCHUNK, CH)
    dst = jnp.concatenate(
        [ei[1], jnp.full((EPAD - E,), N, jnp.int32)]).reshape(NCHUNK, CH)
    return src, dst


def kernel(x_Sestoj, x_Drevo, edge_index_et0, edge_index_et1, edge_index_et2,
           proj_W_Sestoj, proj_b_Sestoj, proj_W_Drevo, proj_b_Drevo,
           conv0_et0_W, conv0_et0_root, conv0_et0_b,
           conv0_et1_W, conv0_et1_root, conv0_et1_b,
           conv0_et2_W, conv0_et2_root, conv0_et2_b,
           norm0_Sestoj_w, norm0_Sestoj_b, norm0_Drevo_w, norm0_Drevo_b,
           conv1_et0_W, conv1_et0_root, conv1_et0_b,
           conv1_et1_W, conv1_et1_root, conv1_et1_b,
           conv1_et2_W, conv1_et2_root, conv1_et2_b,
           norm1_Sestoj_w, norm1_Sestoj_b, norm1_Drevo_w, norm1_Drevo_b,
           lin_W, lin_b):
    conv = [
        [(conv0_et0_W, conv0_et0_root, conv0_et0_b),
         (conv0_et1_W, conv0_et1_root, conv0_et1_b),
         (conv0_et2_W, conv0_et2_root, conv0_et2_b)],
        [(conv1_et0_W, conv1_et0_root, conv1_et0_b),
         (conv1_et1_W, conv1_et1_root, conv1_et1_b),
         (conv1_et2_W, conv1_et2_root, conv1_et2_b)],
    ]
    norm = [
        {"S": (norm0_Sestoj_w, norm0_Sestoj_b), "D": (norm0_Drevo_w, norm0_Drevo_b)},
        {"S": (norm1_Sestoj_w, norm1_Sestoj_b), "D": (norm1_Drevo_w, norm1_Drevo_b)},
    ]

    src0, dst0 = _pad_edges(edge_index_et0)
    src1, dst1 = _pad_edges(edge_index_et1)
    src2, dst2 = _pad_edges(edge_index_et2)

    zeros = jnp.zeros((RPT, H), jnp.float32)
    zeros1d = jnp.zeros((NP,), jnp.float32)

    xp_s = jnp.pad(x_Sestoj, ((0, NP - N), (0, 0)))
    xp_d = jnp.pad(x_Drevo, ((0, NP - N), (0, 0)))
    xs = _proj(xp_s, proj_W_Sestoj, proj_b_Sestoj)
    xd = _proj(xp_d, proj_W_Drevo, proj_b_Drevo)

    cnt_all = _counts(dst0, dst1, dst2, zeros1d).reshape(3, NW, NP)
    cnt = [cnt_all[0], cnt_all[1], cnt_all[2]]

    for l in range(2):
        s0, s1, s2 = _agg3(xs, xd, src0, dst0, src1, dst1, src2, dst2, zeros)
        (w0, r0, b0), (w1, r1, b1), (w2, r2, b2) = conv[l]
        nw_s, nb_s = norm[l]["S"]
        nw_d, nb_d = norm[l]["D"]
        new_xd = _combine([s0], [cnt[0]], xd, [w0], [r0], [b0], nw_d, nb_d)
        new_xs = _combine([s1, s2], [cnt[1], cnt[2]], xs,
                          [w1, w2], [r1, r2], [b1, b2], nw_s, nb_s)
        xs, xd = new_xs, new_xd

    return _final(xs, lin_W, lin_b)[:N]


# speedup vs baseline: 1.1036x; 1.1036x over previous
import functools

import jax
import jax.numpy as jnp
from jax import lax
from jax.experimental import pallas as pl
from jax.experimental.pallas import tpu as pltpu
from jax.experimental.pallas import tpu_sc as plsc

N = 10000
E = 320000
D = 128
H = 128
OUT = 64

NC = 2
NS = 16
NW = NC * NS
CH = 64
KCH = 160
PH = 4
KPH = KCH // PH
SEG = 20
NBUF = 4
NCHUNK = NW * KCH
EPAD = NCHUNK * CH
NP = 10240
RPT = NP // NS

BLK = 640
GRID = NP // BLK


def _proj_body(x_ref, w_ref, b_ref, o_ref):
    h = jnp.dot(x_ref[...], w_ref[...], preferred_element_type=jnp.float32)
    o_ref[...] = jnp.maximum(h + b_ref[...], 0.0)


def _proj(x, w, b):
    return pl.pallas_call(
        _proj_body,
        grid=(GRID,),
        in_specs=[
            pl.BlockSpec((BLK, D), lambda i: (i, 0)),
            pl.BlockSpec((D, H), lambda i: (0, 0)),
            pl.BlockSpec((1, H), lambda i: (0, 0)),
        ],
        out_specs=pl.BlockSpec((BLK, H), lambda i: (i, 0)),
        out_shape=jax.ShapeDtypeStruct((NP, H), jnp.float32),
    )(x, w, b.reshape(1, H))


def _combine_body(k, *refs):
    s_refs = refs[0:k]
    c_refs = refs[k:2 * k]
    x_ref = refs[2 * k]
    w_refs = refs[2 * k + 1:3 * k + 1]
    r_refs = refs[3 * k + 1:4 * k + 1]
    b_refs = refs[4 * k + 1:5 * k + 1]
    nw_ref = refs[5 * k + 1]
    nb_ref = refs[5 * k + 2]
    o_ref = refs[5 * k + 3]

    h = jnp.zeros((BLK, H), jnp.float32)
    r = jnp.zeros((H, H), jnp.float32)
    for e in range(k):
        s = s_refs[e][0] + s_refs[e][1]
        c = jnp.sum(c_refs[e][...], axis=0)[:, None]
        a = s / jnp.maximum(c, 1.0)
        h = h + jnp.dot(a, w_refs[e][...], preferred_element_type=jnp.float32)
        h = h + b_refs[e][...]
        r = r + r_refs[e][...]
    h = (h + jnp.dot(x_ref[...], r, preferred_element_type=jnp.float32)) / float(k)
    mu = jnp.mean(h, axis=-1, keepdims=True)
    var = jnp.mean((h - mu) ** 2, axis=-1, keepdims=True)
    h = (h - mu) * lax.rsqrt(var + 1e-5) * nw_ref[...] + nb_ref[...]
    o_ref[...] = jnp.maximum(h, 0.0)


def _combine(sums, cnts, x, ws, rs, bs, nw, nb):
    k = len(sums)
    in_specs = (
        [pl.BlockSpec((NC, BLK, H), lambda i: (0, i, 0))] * k
        + [pl.BlockSpec((NW, BLK), lambda i: (0, i))] * k
        + [pl.BlockSpec((BLK, H), lambda i: (i, 0))]
        + [pl.BlockSpec((H, H), lambda i: (0, 0))] * (2 * k)
        + [pl.BlockSpec((1, H), lambda i: (0, 0))] * (k + 2)
    )
    args = (list(sums) + list(cnts) + [x] + list(ws) + list(rs)
            + [b.reshape(1, H) for b in bs] + [nw.reshape(1, H), nb.reshape(1, H)])
    return pl.pallas_call(
        functools.partial(_combine_body, k),
        grid=(GRID,),
        in_specs=in_specs,
        out_specs=pl.BlockSpec((BLK, H), lambda i: (i, 0)),
        out_shape=jax.ShapeDtypeStruct((NP, H), jnp.float32),
    )(*args)


def _final_body(x_ref, w_ref, b_ref, o_ref):
    h = jnp.dot(x_ref[...], w_ref[...], preferred_element_type=jnp.float32)
    o_ref[...] = h + b_ref[...]


def _final(x, w, b):
    return pl.pallas_call(
        _final_body,
        grid=(GRID,),
        in_specs=[
            pl.BlockSpec((BLK, H), lambda i: (i, 0)),
            pl.BlockSpec((H, OUT), lambda i: (0, 0)),
            pl.BlockSpec((1, OUT), lambda i: (0, 0)),
        ],
        out_specs=pl.BlockSpec((BLK, OUT), lambda i: (i, 0)),
        out_shape=jax.ShapeDtypeStruct((NP, OUT), jnp.float32),
    )(x, w, b.reshape(1, OUT))


@functools.cache
def _mesh():
    return plsc.VectorSubcoreMesh(
        core_axis_name="c", subcore_axis_name="s", num_cores=NC, num_subcores=NS)


def _agg_body(x_hbm, src_hbm, dst_hbm, zeros_hbm, out_hbm,
              idx_s, idx_d, buf_a, buf_b, buf_c, buf_d, acc,
              sem_a, sem_b, sem_c, sem_d):
    bufs = (buf_a, buf_b, buf_c, buf_d)
    sems = (sem_a, sem_b, sem_c, sem_d)
    cid = lax.axis_index("c")
    sid = lax.axis_index("s")
    wid = sid * NC + cid
    r0 = sid * RPT
    c0 = wid * KCH

    pltpu.sync_copy(zeros_hbm, acc.at[pl.ds(r0, RPT)])
    plsc.subcore_barrier()

    for p in range(PH):
        base = c0 + p * KPH
        pltpu.sync_copy(src_hbm.at[pl.ds(base, KPH)], idx_s)
        pltpu.sync_copy(dst_hbm.at[pl.ds(base, KPH)], idx_d)

        def trip(seg, carry):
            b = seg * SEG
            q = [pltpu.async_copy(x_hbm.at[idx_s.at[b + n]], bufs[n], sems[n])
                 for n in range(NBUF - 1)]
            for j in range(SEG):
                if j + NBUF - 1 < SEG:
                    q.append(pltpu.async_copy(
                        x_hbm.at[idx_s.at[b + j + NBUF - 1]],
                        bufs[(j + NBUF - 1) % NBUF], sems[(j + NBUF - 1) % NBUF]))
                q.pop(0).wait()
                pltpu.sync_copy(bufs[j % NBUF], acc.at[idx_d.at[b + j]], add=True)
            return carry

        lax.fori_loop(0, KPH // SEG, trip, 0)

    plsc.subcore_barrier()
    pltpu.sync_copy(acc.at[pl.ds(r0, RPT)], out_hbm.at[cid, pl.ds(r0, RPT)])


def _agg(x, srcp, dstp, zeros):
    return pl.kernel(
        _agg_body,
        out_type=jax.ShapeDtypeStruct((NC, NP, H), jnp.float32),
        mesh=_mesh(),
        scratch_types=[
            pltpu.VMEM((KPH, CH), jnp.int32),
            pltpu.VMEM((KPH, CH), jnp.int32),
            pltpu.VMEM((CH, H), jnp.float32),
            pltpu.VMEM((CH, H), jnp.float32),
            pltpu.VMEM((CH, H), jnp.float32),
            pltpu.VMEM((CH, H), jnp.float32),
            pltpu.VMEM_SHARED((NP, H), jnp.float32),
            pltpu.SemaphoreType.DMA,
            pltpu.SemaphoreType.DMA,
            pltpu.SemaphoreType.DMA,
            pltpu.SemaphoreType.DMA,
        ],
    )(x, srcp, dstp, zeros)


def _counts_body(d0_hbm, d1_hbm, d2_hbm, zeros_hbm, cnt_hbm, idx_d, cnt, sem):
    cid = lax.axis_index("c")
    sid = lax.axis_index("s")
    wid = sid * NC + cid
    c0 = wid * KCH
    ones = jnp.ones((16,), jnp.float32)

    for rel, d_hbm in enumerate((d0_hbm, d1_hbm, d2_hbm)):
        pltpu.sync_copy(zeros_hbm, cnt)
        pltpu.sync_copy(d_hbm.at[pl.ds(c0, KCH)], idx_d)

        def body(i, carry):
            for j in range(CH // 16):
                idx = idx_d[i, pl.ds(j * 16, 16)]
                plsc.addupdate_scatter(cnt, [idx], ones)
            return carry

        lax.fori_loop(0, KCH, body, 0)
        pltpu.sync_copy(cnt, cnt_hbm.at[pl.ds((rel * NW + wid) * NP, NP)])


def _counts(dst0, dst1, dst2, zeros1d):
    return pl.kernel(
        _counts_body,
        out_type=jax.ShapeDtypeStruct((3 * NW * NP,), jnp.float32),
        mesh=_mesh(),
        compiler_params=pltpu.CompilerParams(needs_layout_passes=False),
        scratch_types=[
            pltpu.VMEM((KCH, CH), jnp.int32),
            pltpu.VMEM((NP,), jnp.float32),
            pltpu.SemaphoreType.DMA,
        ],
    )(dst0, dst1, dst2, zeros1d)


def _pad_edges(ei):
    src = jnp.concatenate(
        [ei[0], jnp.zeros((EPAD - E,), jnp.int32)]).reshape(NCHUNK, CH)
    dst = jnp.concatenate(
        [ei[1], jnp.full((EPAD - E,), N, jnp.int32)]).reshape(NCHUNK, CH)
    return src, dst


def kernel(x_Sestoj, x_Drevo, edge_index_et0, edge_index_et1, edge_index_et2,
           proj_W_Sestoj, proj_b_Sestoj, proj_W_Drevo, proj_b_Drevo,
           conv0_et0_W, conv0_et0_root, conv0_et0_b,
           conv0_et1_W, conv0_et1_root, conv0_et1_b,
           conv0_et2_W, conv0_et2_root, conv0_et2_b,
           norm0_Sestoj_w, norm0_Sestoj_b, norm0_Drevo_w, norm0_Drevo_b,
           conv1_et0_W, conv1_et0_root, conv1_et0_b,
           conv1_et1_W, conv1_et1_root, conv1_et1_b,
           conv1_et2_W, conv1_et2_root, conv1_et2_b,
           norm1_Sestoj_w, norm1_Sestoj_b, norm1_Drevo_w, norm1_Drevo_b,
           lin_W, lin_b):
    conv = [
        [(conv0_et0_W, conv0_et0_root, conv0_et0_b),
         (conv0_et1_W, conv0_et1_root, conv0_et1_b),
         (conv0_et2_W, conv0_et2_root, conv0_et2_b)],
        [(conv1_et0_W, conv1_et0_root, conv1_et0_b),
         (conv1_et1_W, conv1_et1_root, conv1_et1_b),
         (conv1_et2_W, conv1_et2_root, conv1_et2_b)],
    ]
    norm = [
        {"S": (norm0_Sestoj_w, norm0_Sestoj_b), "D": (norm0_Drevo_w, norm0_Drevo_b)},
        {"S": (norm1_Sestoj_w, norm1_Sestoj_b), "D": (norm1_Drevo_w, norm1_Drevo_b)},
    ]

    src0, dst0 = _pad_edges(edge_index_et0)
    src1, dst1 = _pad_edges(edge_index_et1)
    src2, dst2 = _pad_edges(edge_index_et2)

    zeros = jnp.zeros((RPT, H), jnp.float32)
    zeros1d = jnp.zeros((NP,), jnp.float32)

    xp_s = jnp.pad(x_Sestoj, ((0, NP - N), (0, 0)))
    xp_d = jnp.pad(x_Drevo, ((0, NP - N), (0, 0)))
    xs = _proj(xp_s, proj_W_Sestoj, proj_b_Sestoj)
    xd = _proj(xp_d, proj_W_Drevo, proj_b_Drevo)

    cnt_all = _counts(dst0, dst1, dst2, zeros1d).reshape(3, NW, NP)
    cnt = [cnt_all[0], cnt_all[1], cnt_all[2]]

    for l in range(2):
        s0 = _agg(xs, src0, dst0, zeros)
        s1 = _agg(xd, src1, dst1, zeros)
        s2 = _agg(xs, src2, dst2, zeros)
        (w0, r0, b0), (w1, r1, b1), (w2, r2, b2) = conv[l]
        nw_s, nb_s = norm[l]["S"]
        nw_d, nb_d = norm[l]["D"]
        new_xd = _combine([s0], [cnt[0]], xd, [w0], [r0], [b0], nw_d, nb_d)
        new_xs = _combine([s1, s2], [cnt[1], cnt[2]], xs,
                          [w1, w2], [r1, r2], [b1, b2], nw_s, nb_s)
        xs, xd = new_xs, new_xd

    return _final(xs, lin_W, lin_b)[:N]
